# Initial kernel scaffold; baseline (speedup 1.0000x reference)
import jax
import jax.numpy as jnp
from jax.experimental import pallas as pl


def kernel(x, gate_W, gate_b, expert_W, expert_b):
    raise NotImplementedError("write your pallas kernel here")


# speedup vs baseline: 1.0933x; 1.0933x over previous
import functools

import jax
import jax.numpy as jnp
from jax.experimental import pallas as pl
from jax.experimental.pallas import tpu as pltpu


def _dense_moe_body(x_ref, gw_ref, gb_ref, w_ref, b_ref, out_ref):
    e = pl.program_id(1)
    x = x_ref[...]
    logits = jnp.dot(x, gw_ref[...].T, preferred_element_type=jnp.float32)
    logits = logits + gb_ref[...]
    m1 = jnp.max(logits, axis=-1, keepdims=True)
    masked = jnp.where(logits >= m1, -jnp.inf, logits)
    m2 = jnp.max(masked, axis=-1, keepdims=True)
    top2 = logits >= m2
    ecol = jax.lax.broadcasted_iota(jnp.int32, top2.shape, 1)
    mcol = jnp.sum(jnp.where((ecol == e) & top2, 1.0, 0.0), axis=1,
                   keepdims=True)
    y = jax.lax.dot_general(
        x.astype(jnp.bfloat16), w_ref[0],
        (((1,), (1,)), ((), ())), preferred_element_type=jnp.float32)
    y = y + b_ref[0]
    contrib = mcol * y

    @pl.when(e == 0)
    def _():
        out_ref[...] = contrib

    @pl.when(e != 0)
    def _():
        out_ref[...] += contrib


def kernel(x, gate_W, gate_b, expert_W, expert_b):
    orig_shape = x.shape
    D = x.shape[-1]
    M = x.size // D
    E, O = expert_W.shape[0], expert_W.shape[1]
    xf = x.reshape(M, D)
    wq = expert_W.astype(jnp.bfloat16)
    gb2 = gate_b.reshape(1, E)
    eb3 = expert_b.reshape(E, 1, O)

    TB = min(1024, M)
    grid = (M // TB, E)

    out = pl.pallas_call(
        _dense_moe_body,
        grid=grid,
        in_specs=[
            pl.BlockSpec((TB, D), lambda i, e: (i, 0)),
            pl.BlockSpec((E, D), lambda i, e: (0, 0)),
            pl.BlockSpec((1, E), lambda i, e: (0, 0)),
            pl.BlockSpec((1, O, D), lambda i, e: (e, 0, 0)),
            pl.BlockSpec((1, 1, O), lambda i, e: (e, 0, 0)),
        ],
        out_specs=pl.BlockSpec((TB, O), lambda i, e: (i, 0)),
        out_shape=jax.ShapeDtypeStruct((M, O), jnp.float32),
        compiler_params=pltpu.CompilerParams(
            dimension_semantics=("parallel", "arbitrary")),
    )(xf, gate_W, gb2, wq, eb3)
    return out.reshape(orig_shape[:-1] + (O,))


# speedup vs baseline: 1.4379x; 1.3151x over previous
import functools

import jax
import jax.numpy as jnp
from jax import lax
from jax.experimental import pallas as pl
from jax.experimental.pallas import tpu as pltpu
from jax.experimental.pallas import tpu_sc as plsc

NE = 8
NC = 2
NS = 16
TB = 1024
BS = 256


def _gate_body(x_ref, gw_ref, gb_ref, e12_ref, r12_ref, cnt_ref, carry):
    i = pl.program_id(0)
    nb = pl.num_programs(0)

    @pl.when(i == 0)
    def _():
        carry[...] = jnp.zeros_like(carry)

    x = x_ref[...]
    logits = jnp.dot(x, gw_ref[...].T, preferred_element_type=jnp.float32)
    logits = logits + gb_ref[...]
    m1 = jnp.max(logits, axis=-1, keepdims=True)
    l2 = jnp.where(logits >= m1, -jnp.inf, logits)
    m2 = jnp.max(l2, axis=-1, keepdims=True)
    top2 = logits >= m2
    eids = lax.broadcasted_iota(jnp.int32, logits.shape, 1)
    big = jnp.int32(1 << 20)
    e1 = jnp.min(jnp.where(logits >= m1, eids, big), axis=-1, keepdims=True)
    e2 = jnp.min(jnp.where(top2 & (logits < m1), eids, big), axis=-1,
                 keepdims=True)

    maskf = top2.astype(jnp.float32)
    ri = lax.broadcasted_iota(jnp.int32, (TB, TB), 0)
    ci = lax.broadcasted_iota(jnp.int32, (TB, TB), 1)
    tril = jnp.where(ri > ci, 1.0, 0.0)
    excl = jnp.dot(tril, maskf, preferred_element_type=jnp.float32)
    rank_mat = carry[...] + excl
    r1 = jnp.sum(jnp.where(eids == e1, rank_mat, 0.0), axis=1, keepdims=True)
    r2 = jnp.sum(jnp.where(eids == e2, rank_mat, 0.0), axis=1, keepdims=True)

    e12_ref[...] = jnp.concatenate([e1, e2], axis=1)
    r12_ref[...] = jnp.concatenate([r1, r2], axis=1).astype(jnp.int32)

    new_carry = carry[...] + jnp.sum(maskf, axis=0, keepdims=True)
    carry[...] = new_carry

    @pl.when(i == nb - 1)
    def _():
        cnt_ref[...] = new_carry.astype(jnp.int32)


def _gating(xf, gate_W, gb2):
    M, D = xf.shape
    return pl.pallas_call(
        _gate_body,
        grid=(M // TB,),
        in_specs=[
            pl.BlockSpec((TB, D), lambda i: (i, 0)),
            pl.BlockSpec((NE, D), lambda i: (0, 0)),
            pl.BlockSpec((1, NE), lambda i: (0, 0)),
        ],
        out_specs=[
            pl.BlockSpec((TB, 2), lambda i: (i, 0)),
            pl.BlockSpec((TB, 2), lambda i: (i, 0)),
            pl.BlockSpec((1, NE), lambda i: (0, 0)),
        ],
        out_shape=[
            jax.ShapeDtypeStruct((M, 2), jnp.int32),
            jax.ShapeDtypeStruct((M, 2), jnp.int32),
            jax.ShapeDtypeStruct((1, NE), jnp.int32),
        ],
        scratch_shapes=[pltpu.VMEM((1, NE), jnp.float32)],
        compiler_params=pltpu.CompilerParams(
            dimension_semantics=("arbitrary",)),
    )(xf, gate_W, gb2)


def _make_dispatch(M, D, S_pad, n_chunk):
    mesh = plsc.VectorSubcoreMesh(core_axis_name="c", subcore_axis_name="s",
                                  num_cores=NC, num_subcores=NS)
    nw = NC * NS
    per_w = M // nw
    C = per_w // n_chunk

    @functools.partial(
        pl.kernel, mesh=mesh,
        out_type=jax.ShapeDtypeStruct((S_pad, D), jnp.float32),
        scratch_types=[
            pltpu.VMEM((C, D), jnp.float32),
            pltpu.VMEM((C,), jnp.int32),
            pltpu.VMEM((C,), jnp.int32),
        ],
    )
    def dispatch(x_hbm, s1_hbm, s2_hbm, xs_hbm, xbuf, i1, i2):
        wid = lax.axis_index("s") * NC + lax.axis_index("c")
        for c in range(n_chunk):
            base = wid * per_w + c * C
            pltpu.sync_copy(x_hbm.at[pl.ds(base, C)], xbuf)
            pltpu.sync_copy(s1_hbm.at[pl.ds(base, C)], i1)
            pltpu.sync_copy(s2_hbm.at[pl.ds(base, C)], i2)
            pltpu.sync_copy(xbuf, xs_hbm.at[i1])
            pltpu.sync_copy(xbuf, xs_hbm.at[i2])

    return dispatch


def _gmm_body(be_ref, xs_ref, w_ref, b_ref, y_ref):
    y = lax.dot_general(
        xs_ref[...].astype(jnp.bfloat16), w_ref[0],
        (((1,), (1,)), ((), ())), preferred_element_type=jnp.float32)
    y_ref[...] = y + b_ref[0]


def _grouped_matmul(xs, wq, eb3, be):
    S_pad, D = xs.shape
    O = wq.shape[1]
    nb = S_pad // BS
    grid_spec = pltpu.PrefetchScalarGridSpec(
        num_scalar_prefetch=1,
        grid=(nb,),
        in_specs=[
            pl.BlockSpec((BS, D), lambda i, be: (i, 0)),
            pl.BlockSpec((1, O, D), lambda i, be: (be[i], 0, 0)),
            pl.BlockSpec((1, 1, O), lambda i, be: (be[i], 0, 0)),
        ],
        out_specs=pl.BlockSpec((BS, O), lambda i, be: (i, 0)),
    )
    return pl.pallas_call(
        _gmm_body,
        grid_spec=grid_spec,
        out_shape=jax.ShapeDtypeStruct((S_pad, O), jnp.float32),
        compiler_params=pltpu.CompilerParams(
            dimension_semantics=("arbitrary",)),
    )(be, xs, wq, eb3)


def _make_combine(M, O, S_pad, n_chunk):
    mesh = plsc.VectorSubcoreMesh(core_axis_name="c", subcore_axis_name="s",
                                  num_cores=NC, num_subcores=NS)
    nw = NC * NS
    per_w = M // nw
    C = per_w // n_chunk
    L = 16

    @functools.partial(
        pl.kernel, mesh=mesh,
        out_type=jax.ShapeDtypeStruct((M, O), jnp.float32),
        scratch_types=[
            pltpu.VMEM((C, O), jnp.float32),
            pltpu.VMEM((C, O), jnp.float32),
            pltpu.VMEM((C,), jnp.int32),
            pltpu.VMEM((C,), jnp.int32),
        ],
    )
    def combine(ys_hbm, s1_hbm, s2_hbm, out_hbm, b1, b2, i1, i2):
        wid = lax.axis_index("s") * NC + lax.axis_index("c")
        for c in range(n_chunk):
            base = wid * per_w + c * C
            pltpu.sync_copy(s1_hbm.at[pl.ds(base, C)], i1)
            pltpu.sync_copy(s2_hbm.at[pl.ds(base, C)], i2)
            pltpu.sync_copy(ys_hbm.at[i1], b1)
            pltpu.sync_copy(ys_hbm.at[i2], b2)

            def add_cols(j, _):
                for r in range(C):
                    b1[r, pl.ds(j * L, L)] = (b1[r, pl.ds(j * L, L)]
                                              + b2[r, pl.ds(j * L, L)])
                return 0

            lax.fori_loop(0, O // L, add_cols, 0)
            pltpu.sync_copy(b1, out_hbm.at[pl.ds(base, C)])

    return combine


def kernel(x, gate_W, gate_b, expert_W, expert_b):
    orig_shape = x.shape
    D = x.shape[-1]
    M = x.size // D
    O = expert_W.shape[1]
    xf = x.reshape(M, D)
    gb2 = gate_b.reshape(1, NE)
    wq = expert_W.astype(jnp.bfloat16)
    eb3 = expert_b.reshape(NE, 1, O)

    nb_max = M * 2 // BS + (NE - 1)
    S_pad = nb_max * BS

    e12, r12, cnt = _gating(xf, gate_W, gb2)

    counts = cnt[0]
    padded = ((counts + BS - 1) // BS) * BS
    poff = jnp.concatenate([jnp.zeros((1,), jnp.int32),
                            jnp.cumsum(padded)[:-1].astype(jnp.int32)])
    slot = jnp.take(poff, e12, axis=0) + r12
    s1 = slot[:, 0]
    s2 = slot[:, 1]
    starts = jnp.arange(nb_max, dtype=jnp.int32) * BS
    be = (jnp.sum(starts[:, None] >= poff[None, :], axis=1) - 1).astype(
        jnp.int32)
    be = jnp.clip(be, 0, NE - 1)

    xs = _make_dispatch(M, D, S_pad, n_chunk=8)(xf, s1, s2)
    ys = _grouped_matmul(xs, wq, eb3, be)
    out = _make_combine(M, O, S_pad, n_chunk=16)(ys, s1, s2)
    return out.reshape(orig_shape[:-1] + (O,))
